# baseline (speedup 1.0000x reference)
import functools

import jax
import jax.numpy as jnp
from jax import lax
from jax.experimental import pallas as pl
from jax.experimental.pallas import tpu as pltpu
from jax.experimental.pallas import tpu_sc as plsc

T_DIM, H_DIM, W_DIM, EMBED_DIM = 16, 24, 24, 384
BATCH = 8
L = 16
NWORK = 32
ROWS = BATCH * T_DIM * H_DIM * W_DIM
GROUPS = ROWS // W_DIM

M_TC = 56
T_BLK = 8
TC_GROUPS = M_TC * H_DIM
TC_ROWS = TC_GROUPS * W_DIM

SC_GROUPS = GROUPS - TC_GROUPS
GPW = SC_GROUPS // NWORK
GPC = 2
CHUNKS = GPW // GPC
CROWS = GPC * W_DIM
NBUF = 4
LEAD = NBUF // 2
NCOL = EMBED_DIM // L

_MID = CHUNKS - 2 * LEAD
_MID_DYN = _MID - _MID % NBUF


def _in_start(x_hbm, buf, sems, slot, k, base_row):
    pltpu.async_copy(
        x_hbm.at[pl.ds(base_row + k * CROWS, CROWS), :], buf.at[slot], sems[slot]
    )


def _in_wait(x_hbm, buf, sems, slot, base_row):
    pltpu.make_async_copy(
        x_hbm.at[pl.ds(base_row, CROWS), :], buf.at[slot], sems[slot]
    ).wait()


def _out_start(out_hbm, buf, sems, slot, k, base_row):
    pltpu.async_copy(
        buf.at[slot], out_hbm.at[pl.ds(base_row + k * CROWS, CROWS), :],
        sems[NBUF + slot],
    )


def _out_wait(out_hbm, buf, sems, slot, base_row):
    pltpu.make_async_copy(
        buf.at[slot], out_hbm.at[pl.ds(base_row, CROWS), :], sems[NBUF + slot]
    ).wait()


def _compute(buf, tv, hv, wv, slot, k, base_group):
    gid0 = base_group + k * GPC
    ts, hs = [], []
    for g in range(GPC):
        rem = lax.rem(gid0 + g, T_DIM * H_DIM)
        ts.append(lax.div(rem, H_DIM))
        hs.append(lax.rem(rem, H_DIM))

    def col_body(c, carry):
        off = pl.ds(c * L, L)
        wcol = [wv[w, off] for w in range(W_DIM)]
        for g in range(GPC):
            th = tv[ts[g], off] * hv[hs[g], off]
            for w in range(W_DIM):
                r = g * W_DIM + w
                buf[slot, r, off] = buf[slot, r, off] + th * wcol[w]
        return carry

    lax.fori_loop(0, NCOL, col_body, 0)


def _sc_tail(xr, t_embed, h_embed, w_embed):
    mesh = plsc.VectorSubcoreMesh(core_axis_name="c", subcore_axis_name="s")

    @functools.partial(
        pl.kernel,
        mesh=mesh,
        out_type=jax.ShapeDtypeStruct((ROWS, EMBED_DIM), jnp.float32),
        scratch_types=[
            pltpu.VMEM((NBUF, CROWS, EMBED_DIM), jnp.float32),
            pltpu.VMEM((T_DIM, EMBED_DIM), jnp.float32),
            pltpu.VMEM((H_DIM, EMBED_DIM), jnp.float32),
            pltpu.VMEM((W_DIM, EMBED_DIM), jnp.float32),
        ] + [pltpu.SemaphoreType.DMA] * (2 * NBUF),
    )
    def sc_add(x_hbm, t_hbm, h_hbm, w_hbm, out_hbm, buf, tv, hv, wv, *sems):
        wid = lax.axis_index("s") * 2 + lax.axis_index("c")
        base_group = TC_GROUPS + wid * GPW
        base_row = base_group * W_DIM
        pltpu.sync_copy(t_hbm, tv)
        pltpu.sync_copy(h_hbm, hv)
        pltpu.sync_copy(w_hbm, wv)

        def body(k, slot, nslot, first, last):
            if not first:
                _out_wait(out_hbm, buf, sems, nslot, base_row)
            if not last:
                _in_start(x_hbm, buf, sems, nslot, k + LEAD, base_row)
            _in_wait(x_hbm, buf, sems, slot, base_row)
            _compute(buf, tv, hv, wv, slot, k, base_group)
            _out_start(out_hbm, buf, sems, slot, k, base_row)

        for k in range(LEAD):
            _in_start(x_hbm, buf, sems, k, k, base_row)
        for k in range(LEAD):
            body(k, k % NBUF, (k + LEAD) % NBUF, True, False)

        def mid(m, carry):
            k0 = LEAD + m * NBUF
            for j in range(NBUF):
                body(k0 + j, (LEAD + j) % NBUF, j % NBUF, False, False)
            return carry

        lax.fori_loop(0, _MID_DYN // NBUF, mid, 0)

        for k in range(LEAD + _MID_DYN, CHUNKS - LEAD):
            body(k, k % NBUF, (k + LEAD) % NBUF, False, False)
        for k in range(CHUNKS - LEAD, CHUNKS):
            body(k, k % NBUF, (k + LEAD) % NBUF, False, True)
        for k in range(CHUNKS - NBUF + LEAD, CHUNKS):
            _out_wait(out_hbm, buf, sems, k % NBUF, base_row)

    return sc_add(xr, t_embed, h_embed, w_embed)


def _tc_body(t_ref, h_ref, w_ref, x_ref, o_ref):
    t = t_ref[0]
    h = h_ref[...]
    w = w_ref[...]
    th = t[:, None, :] * h[None, :, :]
    pos = th[:, :, None, :] * w[None, None, :, :]
    o_ref[...] = x_ref[...] + pos


def _tc_head(x_full, t_embed, h_embed, w_embed):
    tr = t_embed.reshape(T_DIM // T_BLK, T_BLK, EMBED_DIM)
    grid = (M_TC // T_BLK,)
    return pl.pallas_call(
        _tc_body,
        grid=grid,
        in_specs=[
            pl.BlockSpec((1, T_BLK, EMBED_DIM), lambda i: (i % (T_DIM // T_BLK), 0, 0)),
            pl.BlockSpec((H_DIM, EMBED_DIM), lambda i: (0, 0)),
            pl.BlockSpec((W_DIM, EMBED_DIM), lambda i: (0, 0)),
            pl.BlockSpec((T_BLK, H_DIM, W_DIM, EMBED_DIM), lambda i: (i, 0, 0, 0)),
        ],
        out_specs=pl.BlockSpec((T_BLK, H_DIM, W_DIM, EMBED_DIM), lambda i: (i, 0, 0, 0)),
        out_shape=jax.ShapeDtypeStruct((M_TC, H_DIM, W_DIM, EMBED_DIM), jnp.float32),
    )(tr, h_embed, w_embed, x_full)


def kernel(x, t_embed, h_embed, w_embed):
    xr = x.reshape(ROWS, EMBED_DIM)
    x4 = x.reshape(BATCH * T_DIM, H_DIM, W_DIM, EMBED_DIM)
    out_sc = _sc_tail(xr, t_embed, h_embed, w_embed)
    out_tc = _tc_head(x4, t_embed, h_embed, w_embed).reshape(TC_ROWS, EMBED_DIM)
    out = lax.dynamic_update_slice(out_sc, out_tc, (0, 0))
    return out.reshape(x.shape)


# speedup vs baseline: 1.2076x; 1.2076x over previous
import functools

import jax
import jax.numpy as jnp
from jax import lax
from jax.experimental import pallas as pl
from jax.experimental.pallas import tpu as pltpu
from jax.experimental.pallas import tpu_sc as plsc

T_DIM, H_DIM, W_DIM, EMBED_DIM = 16, 24, 24, 384
BATCH = 8
L = 16
NWORK = 32
ROWS = BATCH * T_DIM * H_DIM * W_DIM
GROUPS = ROWS // W_DIM
GPW = GROUPS // NWORK
GPC = 2
CHUNKS = GPW // GPC
CROWS = GPC * W_DIM
NBUF = 5
LEAD = 2
PEEL = NBUF - LEAD
NCOL = EMBED_DIM // L

_MID = (CHUNKS - LEAD) - PEEL
_MID_DYN = _MID - _MID % NBUF


def _in_start(x_hbm, buf, sems, slot, k, base_row):
    pltpu.async_copy(
        x_hbm.at[pl.ds(base_row + k * CROWS, CROWS), :], buf.at[slot], sems[slot]
    )


def _in_wait(x_hbm, buf, sems, slot, base_row):
    pltpu.make_async_copy(
        x_hbm.at[pl.ds(base_row, CROWS), :], buf.at[slot], sems[slot]
    ).wait()


def _out_start(out_hbm, buf, sems, slot, k, base_row):
    pltpu.async_copy(
        buf.at[slot], out_hbm.at[pl.ds(base_row + k * CROWS, CROWS), :],
        sems[NBUF + slot],
    )


def _out_wait(out_hbm, buf, sems, slot, base_row):
    pltpu.make_async_copy(
        buf.at[slot], out_hbm.at[pl.ds(base_row, CROWS), :], sems[NBUF + slot]
    ).wait()


def _compute(buf, tv, hv, wv, slot, k, base_group):
    gid0 = base_group + k * GPC
    ts, hs = [], []
    for g in range(GPC):
        rem = lax.rem(gid0 + g, T_DIM * H_DIM)
        ts.append(lax.div(rem, H_DIM))
        hs.append(lax.rem(rem, H_DIM))

    def col_body(c, carry):
        off = pl.ds(c * L, L)
        wcol = [wv[w, off] for w in range(W_DIM)]
        for g in range(GPC):
            th = tv[ts[g], off] * hv[hs[g], off]
            for w in range(W_DIM):
                r = g * W_DIM + w
                buf[slot, r, off] = buf[slot, r, off] + th * wcol[w]
        return carry

    lax.fori_loop(0, NCOL, col_body, 0)


def kernel(x, t_embed, h_embed, w_embed):
    xr = x.reshape(ROWS, EMBED_DIM)
    mesh = plsc.VectorSubcoreMesh(core_axis_name="c", subcore_axis_name="s")

    @functools.partial(
        pl.kernel,
        mesh=mesh,
        out_type=jax.ShapeDtypeStruct((ROWS, EMBED_DIM), jnp.float32),
        scratch_types=[
            pltpu.VMEM((NBUF, CROWS, EMBED_DIM), jnp.float32),
            pltpu.VMEM((T_DIM, EMBED_DIM), jnp.float32),
            pltpu.VMEM((H_DIM, EMBED_DIM), jnp.float32),
            pltpu.VMEM((W_DIM, EMBED_DIM), jnp.float32),
        ] + [pltpu.SemaphoreType.DMA] * (2 * NBUF),
    )
    def sc_add(x_hbm, t_hbm, h_hbm, w_hbm, out_hbm, buf, tv, hv, wv, *sems):
        wid = lax.axis_index("s") * 2 + lax.axis_index("c")
        base_group = wid * GPW
        base_row = base_group * W_DIM
        pltpu.sync_copy(t_hbm, tv)
        pltpu.sync_copy(h_hbm, hv)
        pltpu.sync_copy(w_hbm, wv)

        def body(k, slot, nslot, first, last):
            if not first:
                _out_wait(out_hbm, buf, sems, nslot, base_row)
            if not last:
                _in_start(x_hbm, buf, sems, nslot, k + LEAD, base_row)
            _in_wait(x_hbm, buf, sems, slot, base_row)
            _compute(buf, tv, hv, wv, slot, k, base_group)
            _out_start(out_hbm, buf, sems, slot, k, base_row)

        for k in range(LEAD):
            _in_start(x_hbm, buf, sems, k, k, base_row)
        for k in range(PEEL):
            body(k, k % NBUF, (k + LEAD) % NBUF, True, False)

        def mid(m, carry):
            k0 = PEEL + m * NBUF
            for j in range(NBUF):
                body(k0 + j, (PEEL + j) % NBUF, j % NBUF, False, False)
            return carry

        lax.fori_loop(0, _MID_DYN // NBUF, mid, 0)

        for k in range(PEEL + _MID_DYN, CHUNKS - LEAD):
            body(k, k % NBUF, (k + LEAD) % NBUF, False, False)
        for k in range(CHUNKS - LEAD, CHUNKS):
            body(k, k % NBUF, (k + LEAD) % NBUF, False, True)
        for k in range(CHUNKS - PEEL, CHUNKS):
            _out_wait(out_hbm, buf, sems, k % NBUF, base_row)

    out = sc_add(xr, t_embed, h_embed, w_embed)
    return out.reshape(x.shape)


# speedup vs baseline: 1.2220x; 1.0119x over previous
import functools

import jax
import jax.numpy as jnp
from jax import lax
from jax.experimental import pallas as pl
from jax.experimental.pallas import tpu as pltpu
from jax.experimental.pallas import tpu_sc as plsc

T_DIM, H_DIM, W_DIM, EMBED_DIM = 16, 24, 24, 384
BATCH = 8
L = 16
NWORK = 32
ROWS = BATCH * T_DIM * H_DIM * W_DIM
GROUPS = ROWS // W_DIM
GPW = GROUPS // NWORK
GPC = 2
CHUNKS = GPW // GPC
CROWS = GPC * W_DIM
NBUF = 4
LEAD = 2
PEEL = NBUF - LEAD
NCOL = EMBED_DIM // L

_MID = (CHUNKS - LEAD) - PEEL
_MID_DYN = _MID - _MID % NBUF


def _in_start(x_hbm, buf, sems, slot, k, base_row):
    pltpu.async_copy(
        x_hbm.at[pl.ds(base_row + k * CROWS, CROWS), :], buf.at[slot], sems[slot]
    )


def _in_wait(x_hbm, buf, sems, slot, base_row):
    pltpu.make_async_copy(
        x_hbm.at[pl.ds(base_row, CROWS), :], buf.at[slot], sems[slot]
    ).wait()


def _out_start(out_hbm, buf, sems, slot, k, base_row):
    pltpu.async_copy(
        buf.at[slot], out_hbm.at[pl.ds(base_row + k * CROWS, CROWS), :],
        sems[NBUF + slot],
    )


def _out_wait(out_hbm, buf, sems, slot, base_row):
    pltpu.make_async_copy(
        buf.at[slot], out_hbm.at[pl.ds(base_row, CROWS), :], sems[NBUF + slot]
    ).wait()


def _compute(buf, tv, hv, wv, slot, k, base_group):
    gid0 = base_group + k * GPC
    ts, hs = [], []
    for g in range(GPC):
        rem = lax.rem(gid0 + g, T_DIM * H_DIM)
        ts.append(lax.div(rem, H_DIM))
        hs.append(lax.rem(rem, H_DIM))

    def col_body(c, carry):
        off = pl.ds(c * L, L)
        wcol = [wv[w, off] for w in range(W_DIM)]
        for g in range(GPC):
            th = tv[ts[g], off] * hv[hs[g], off]
            for w in range(W_DIM):
                r = g * W_DIM + w
                buf[slot, r, off] = buf[slot, r, off] + th * wcol[w]
        return carry

    lax.fori_loop(0, NCOL, col_body, 0)


def kernel(x, t_embed, h_embed, w_embed):
    xr = x.reshape(ROWS, EMBED_DIM)
    mesh = plsc.VectorSubcoreMesh(core_axis_name="c", subcore_axis_name="s")

    @functools.partial(
        pl.kernel,
        mesh=mesh,
        out_type=jax.ShapeDtypeStruct((ROWS, EMBED_DIM), jnp.float32),
        scratch_types=[
            pltpu.VMEM((NBUF, CROWS, EMBED_DIM), jnp.float32),
            pltpu.VMEM((T_DIM, EMBED_DIM), jnp.float32),
            pltpu.VMEM((H_DIM, EMBED_DIM), jnp.float32),
            pltpu.VMEM((W_DIM, EMBED_DIM), jnp.float32),
        ] + [pltpu.SemaphoreType.DMA] * (2 * NBUF),
    )
    def sc_add(x_hbm, t_hbm, h_hbm, w_hbm, out_hbm, buf, tv, hv, wv, *sems):
        wid = lax.axis_index("s") * 2 + lax.axis_index("c")
        base_group = wid * GPW
        base_row = base_group * W_DIM
        pltpu.sync_copy(t_hbm, tv)
        pltpu.sync_copy(h_hbm, hv)
        pltpu.sync_copy(w_hbm, wv)

        def body(k, slot, nslot, first, last):
            if not first:
                _out_wait(out_hbm, buf, sems, nslot, base_row)
            if not last:
                _in_start(x_hbm, buf, sems, nslot, k + LEAD, base_row)
            _in_wait(x_hbm, buf, sems, slot, base_row)
            _compute(buf, tv, hv, wv, slot, k, base_group)
            _out_start(out_hbm, buf, sems, slot, k, base_row)

        for k in range(LEAD):
            _in_start(x_hbm, buf, sems, k, k, base_row)
        for k in range(PEEL):
            body(k, k % NBUF, (k + LEAD) % NBUF, True, False)

        def mid(m, carry):
            k0 = PEEL + m * NBUF
            for j in range(NBUF):
                body(k0 + j, (PEEL + j) % NBUF, j % NBUF, False, False)
            return carry

        lax.fori_loop(0, _MID_DYN // NBUF, mid, 0)

        for k in range(PEEL + _MID_DYN, CHUNKS - LEAD):
            body(k, k % NBUF, (k + LEAD) % NBUF, False, False)
        for k in range(CHUNKS - LEAD, CHUNKS):
            body(k, k % NBUF, (k + LEAD) % NBUF, False, True)
        for k in range(CHUNKS - PEEL, CHUNKS):
            _out_wait(out_hbm, buf, sems, k % NBUF, base_row)

    out = sc_add(xr, t_embed, h_embed, w_embed)
    return out.reshape(x.shape)
